# baseline (speedup 1.0000x reference)
import jax
import jax.numpy as jnp
from jax.experimental import pallas as pl

_B, _T = 4, 8192
_R = 8
_Q = _T // _R
_DM, _KLAT, _DMODEL = 64, 16, 128
_DIN, _DA = 64, 32
_KSEL, _KEFF = 8, 16
_NEG = -1e30


def _body(x_ref, fb_ref, wu_ref, bur_ref, buc_ref, wa_ref, bac_ref, wmat_ref,
          m0r_ref, wsc_ref, bs_ref, pos_ref, wz_ref, bz_ref, wr_ref, br_ref,
          wh_ref, bh_ref, mu_ref, sig_ref, wl_ref, bl_ref, wp_ref, bp_ref,
          y_ref, tok_ref, mem_ref):
    f32 = jnp.float32
    i32 = jnp.int32
    wu = wu_ref[...]
    m0r = m0r_ref[...]
    wsc = wsc_ref[...]
    mwa_c = jnp.sum(wmat_ref[...] * m0r, axis=1, keepdims=True)
    iota_l = jax.lax.broadcasted_iota(i32, (1, _T), 1)

    dn_t = (((0,), (1,)), ((), ()))
    dn_tt = (((0,), (0,)), ((), ()))

    xb = x_ref[0]
    ut = jnp.tanh(
        jax.lax.dot_general(wu, xb, dn_t, preferred_element_type=f32)
        + buc_ref[...])
    at = jnp.tanh(
        jax.lax.dot_general(wa_ref[...], ut, dn_tt, preferred_element_type=f32)
        + mwa_c + bac_ref[...])
    s8 = (jnp.sum(at * wsc, axis=0, keepdims=True)
          + bs_ref[...] + pos_ref[...])
    ig = iota_l
    maxs = jnp.max(s8, keepdims=True)
    p_full = jnp.exp(s8 - maxs)
    sumexp = jnp.sum(p_full, keepdims=True)

    ms = s8
    selmask = jnp.zeros(s8.shape, jnp.bool_)
    pidxs, pjs = [], []
    for _ in range(_KSEL):
        v = jnp.max(ms, keepdims=True)
        pidx = jnp.min(jnp.where(ms == v, ig, _T), keepdims=True)
        pjs.append(jnp.exp(v - maxs) / sumexp)
        pidxs.append(pidx)
        selmask = selmask | (ig == pidx)
        ms = jnp.where(jnp.abs(ig - pidx) <= 1, _NEG, ms)

    p8r = jnp.concatenate(pjs, axis=1)
    i8r = jnp.concatenate(pidxs, axis=1)
    p8c = jnp.concatenate(pjs, axis=0)
    i8c = jnp.concatenate(pidxs, axis=0)

    y_ref[0] = jnp.where(selmask, p_full / sumexp, 0.0)
    sump = jnp.sum(p8r, keepdims=True)
    denom = sump + 1e-8

    before = (p8c > p8r) | ((p8c == p8r) & (i8c < i8r))
    rank = jnp.sum(before.astype(i32), axis=0, keepdims=True)
    k8c = jax.lax.broadcasted_iota(i32, (_KSEL, 1), 0)
    mrank = (rank == k8c).astype(f32)
    svals = jnp.sum(mrank * p8r, axis=1, keepdims=True)
    sidx = jnp.sum(mrank.astype(i32) * i8r, axis=1, keepdims=True)
    t16r = jax.lax.broadcasted_iota(i32, (1, 2 * _KSEL), 1)
    picked = jnp.zeros((1, 2 * _KSEL), jnp.bool_)
    for pidx in pidxs:
        picked = picked | (t16r == pidx)
    free = ~picked
    t16c = jax.lax.broadcasted_iota(i32, (2 * _KSEL, 1), 0)
    free_c = jnp.sum((t16c == t16r).astype(i32)
                     * free.astype(i32), axis=1, keepdims=True)
    bc = jnp.sum(jnp.where((t16c < t16r) & (free_c > 0), 1, 0),
                 axis=0, keepdims=True)
    m2 = ((bc == k8c) & free).astype(i32)
    zidx = jnp.sum(m2 * t16r, axis=1, keepdims=True)
    tii = jnp.concatenate([sidx, zidx], axis=0)
    tv = jnp.concatenate([svals, jnp.zeros((_KSEL, 1), f32)], axis=0)

    sel17 = jnp.concatenate(
        [(tii == iota_l).astype(f32), jnp.ones((1, _T), f32)], axis=0)
    g17 = jnp.dot(sel17, xb, preferred_element_type=f32)
    xg = g17[0:_KEFF, :]
    xmean = g17[_KEFF:_KEFF + 1, :] * (1.0 / _T)

    i8f = i8r.astype(f32)
    cn = jnp.sum(p8r * (i8r <= tii).astype(f32), axis=1,
                 keepdims=True) / denom
    mean_cn = jnp.sum(p8r * (_T - i8f), keepdims=True) / (denom * _T)

    posn = tii.astype(f32) * (1.0 / _T)
    dvec = jnp.concatenate([xg, tv, posn, cn], axis=1)
    mp = jnp.full((1, 1), (_T - 1) / (2.0 * _T), f32)
    dmean = jnp.concatenate(
        [xmean, sump * (1.0 / _T), mp, mean_cn], axis=1)
    c = dvec - dmean
    c = c / (jnp.sqrt(jnp.sum(c * c, axis=1, keepdims=True)) + 1e-6)
    zz = (c - mu_ref[...]) / sig_ref[...]
    lif = jnp.tanh(jnp.dot(zz, wl_ref[...], preferred_element_type=f32)
                   + bl_ref[...])
    lif = lif / (jnp.sqrt(jnp.sum(lif * lif, axis=1, keepdims=True)) + 1e-6)
    tok_ref[0] = (jnp.dot(lif, wp_ref[...], preferred_element_type=f32)
                  + bp_ref[...])

    u8 = jnp.tanh(jnp.dot(xg[0:_KSEL, :], wu, preferred_element_type=f32)
                  + bur_ref[...])
    w8 = tv[0:_KSEL, :] / denom
    ctx = jnp.sum(w8 * u8, axis=0, keepdims=True)
    inp = jnp.concatenate([ctx, fb_ref[0]], axis=1)
    xh = jnp.concatenate([inp, m0r], axis=1)
    zg = jax.nn.sigmoid(jnp.dot(xh, wz_ref[...], preferred_element_type=f32)
                        + bz_ref[...])
    rg = jax.nn.sigmoid(jnp.dot(xh, wr_ref[...], preferred_element_type=f32)
                        + br_ref[...])
    xrh = jnp.concatenate([inp, rg * m0r], axis=1)
    hh = jnp.tanh(jnp.dot(xrh, wh_ref[...], preferred_element_type=f32)
                  + bh_ref[...])
    m1 = (1.0 - zg) * m0r + zg * hh
    mem_ref[0] = jnp.concatenate([m0r, m1], axis=0)


def kernel(x, feedback, params):
    p = params
    B, T, _ = x.shape
    f32 = jnp.float32

    full = lambda shape: pl.BlockSpec(shape, lambda b: tuple(0 for _ in shape))
    in_specs = [
        pl.BlockSpec((1, T, _DIN), lambda b: (b, 0, 0)),
        pl.BlockSpec((1, 1, 1), lambda b: (b, 0, 0)),
        full((_DIN, _DIN)), full((1, _DIN)), full((_DIN, 1)),
        full((_DIN, _DA)), full((_DA, 1)),
        full((_DA, _DM)), full((1, _DM)),
        full((_DA, 1)), full((1, 1)), full((1, T)),
        full((_DM * 2 + 1, _DM)), full((1, _DM)),
        full((_DM * 2 + 1, _DM)), full((1, _DM)),
        full((_DM * 2 + 1, _DM)), full((1, _DM)),
        full((1, _DIN + 3)), full((1, _DIN + 3)),
        full((_DIN + 3, _KLAT)), full((1, _KLAT)),
        full((_KLAT, _DMODEL)), full((1, _DMODEL)),
    ]
    out_specs = (
        pl.BlockSpec((1, 1, T), lambda b: (b, 0, 0)),
        pl.BlockSpec((1, _KEFF, _DMODEL), lambda b: (b, 0, 0)),
        pl.BlockSpec((1, 2, _DM), lambda b: (b, 0, 0)),
    )
    all_y, tokens, mem = pl.pallas_call(
        _body,
        grid=(B,),
        in_specs=in_specs,
        out_specs=out_specs,
        out_shape=(
            jax.ShapeDtypeStruct((B, 1, T), f32),
            jax.ShapeDtypeStruct((B, _KEFF, _DMODEL), f32),
            jax.ShapeDtypeStruct((B, 2, _DM), f32),
        ),
    )(
        x, feedback.reshape(B, 1, 1),
        p['W_u'], p['b_u'].reshape(1, -1), p['b_u'].reshape(-1, 1),
        p['W_a'], p['b_a'].reshape(-1, 1),
        p['W_ma'].T, p['m0'].reshape(1, -1),
        p['w_s'].reshape(-1, 1), p['b_s'].reshape(1, 1),
        p['pos_bias'][:T].reshape(1, -1),
        p['W_z'], p['b_z'].reshape(1, -1),
        p['W_r'], p['b_r'].reshape(1, -1),
        p['W_h'], p['b_h'].reshape(1, -1),
        p['mu'].reshape(1, -1), p['sigma'].reshape(1, -1),
        p['W_lift'], p['b_lift'].reshape(1, -1),
        p['W_proj'], p['b_proj'].reshape(1, -1),
    )
    y_star = all_y[:, 0, :]
    return tokens, y_star, all_y, mem


# speedup vs baseline: 1.0268x; 1.0268x over previous
import jax
import jax.numpy as jnp
from jax.experimental import pallas as pl

_B, _T = 4, 8192
_R = 8
_Q = _T // _R
_DM, _KLAT, _DMODEL = 64, 16, 128
_DIN, _DA = 64, 32
_KSEL, _KEFF = 8, 16
_NEG = -1e30


def _body(x_ref, fb_ref, wu_ref, bur_ref, buc_ref, wa_ref, bac_ref, wmat_ref,
          m0r_ref, wsc_ref, bs_ref, pos_ref, wz_ref, bz_ref, wr_ref, br_ref,
          wh_ref, bh_ref, mu_ref, sig_ref, wl_ref, bl_ref, wp_ref, bp_ref,
          y_ref, tok_ref, mem_ref):
    f32 = jnp.float32
    i32 = jnp.int32
    wu = wu_ref[...]
    m0r = m0r_ref[...]
    wsc = wsc_ref[...]
    mwa_c = jnp.sum(wmat_ref[...] * m0r, axis=1, keepdims=True)
    iota_l = jax.lax.broadcasted_iota(i32, (1, _T), 1)

    dn_t = (((0,), (1,)), ((), ()))
    dn_tt = (((0,), (0,)), ((), ()))

    xb = x_ref[0]
    ut = jnp.tanh(
        jax.lax.dot_general(wu, xb, dn_t, preferred_element_type=f32)
        + buc_ref[...])
    at = jnp.tanh(
        jax.lax.dot_general(wa_ref[...], ut, dn_tt, preferred_element_type=f32)
        + mwa_c + bac_ref[...])
    s8 = jnp.concatenate(
        [jnp.sum(at[:, _Q * i:_Q * (i + 1)] * wsc, axis=0, keepdims=True)
         for i in range(_R)], axis=0) + bs_ref[...] + pos_ref[...]
    ig = (_Q * jax.lax.broadcasted_iota(i32, (_R, _Q), 0)
          + jax.lax.broadcasted_iota(i32, (_R, _Q), 1))
    maxs = jnp.max(s8, keepdims=True)
    p_full = jnp.exp(s8 - maxs)
    sumexp = jnp.sum(p_full, keepdims=True)

    ms = s8
    selmask = jnp.zeros(s8.shape, jnp.bool_)
    pidxs, pjs = [], []
    for _ in range(_KSEL):
        v = jnp.max(ms, keepdims=True)
        pidx = jnp.min(jnp.where(ms == v, ig, _T), keepdims=True)
        pjs.append(jnp.exp(v - maxs) / sumexp)
        pidxs.append(pidx)
        selmask = selmask | (ig == pidx)
        ms = jnp.where(jnp.abs(ig - pidx) <= 1, _NEG, ms)

    p8r = jnp.concatenate(pjs, axis=1)
    i8r = jnp.concatenate(pidxs, axis=1)
    p8c = jnp.concatenate(pjs, axis=0)
    i8c = jnp.concatenate(pidxs, axis=0)

    y_ref[0] = jnp.where(selmask, p_full / sumexp, 0.0)
    sump = jnp.sum(p8r, keepdims=True)
    denom = sump + 1e-8

    before = (p8c > p8r) | ((p8c == p8r) & (i8c < i8r))
    rank = jnp.sum(before.astype(i32), axis=0, keepdims=True)
    k8c = jax.lax.broadcasted_iota(i32, (_KSEL, 1), 0)
    mrank = (rank == k8c).astype(f32)
    svals = jnp.sum(mrank * p8r, axis=1, keepdims=True)
    sidx = jnp.sum(mrank.astype(i32) * i8r, axis=1, keepdims=True)
    t16r = jax.lax.broadcasted_iota(i32, (1, 2 * _KSEL), 1)
    picked = jnp.zeros((1, 2 * _KSEL), jnp.bool_)
    for pidx in pidxs:
        picked = picked | (t16r == pidx)
    free = ~picked
    t16c = jax.lax.broadcasted_iota(i32, (2 * _KSEL, 1), 0)
    free_c = jnp.sum((t16c == t16r).astype(i32)
                     * free.astype(i32), axis=1, keepdims=True)
    bc = jnp.sum(jnp.where((t16c < t16r) & (free_c > 0), 1, 0),
                 axis=0, keepdims=True)
    m2 = ((bc == k8c) & free).astype(i32)
    zidx = jnp.sum(m2 * t16r, axis=1, keepdims=True)
    tii = jnp.concatenate([sidx, zidx], axis=0)
    tv = jnp.concatenate([svals, jnp.zeros((_KSEL, 1), f32)], axis=0)

    sel17 = jnp.concatenate(
        [(tii == iota_l).astype(f32), jnp.ones((1, _T), f32)], axis=0)
    g17 = jnp.dot(sel17, xb, preferred_element_type=f32)
    xg = g17[0:_KEFF, :]
    xmean = g17[_KEFF:_KEFF + 1, :] * (1.0 / _T)

    i8f = i8r.astype(f32)
    cn = jnp.sum(p8r * (i8r <= tii).astype(f32), axis=1,
                 keepdims=True) / denom
    mean_cn = jnp.sum(p8r * (_T - i8f), keepdims=True) / (denom * _T)

    posn = tii.astype(f32) * (1.0 / _T)
    dvec = jnp.concatenate([xg, tv, posn, cn], axis=1)
    mp = jnp.full((1, 1), (_T - 1) / (2.0 * _T), f32)
    dmean = jnp.concatenate(
        [xmean, sump * (1.0 / _T), mp, mean_cn], axis=1)
    c = dvec - dmean
    c = c / (jnp.sqrt(jnp.sum(c * c, axis=1, keepdims=True)) + 1e-6)
    zz = (c - mu_ref[...]) / sig_ref[...]
    lif = jnp.tanh(jnp.dot(zz, wl_ref[...], preferred_element_type=f32)
                   + bl_ref[...])
    lif = lif / (jnp.sqrt(jnp.sum(lif * lif, axis=1, keepdims=True)) + 1e-6)
    tok_ref[0] = (jnp.dot(lif, wp_ref[...], preferred_element_type=f32)
                  + bp_ref[...])

    u8 = jnp.tanh(jnp.dot(xg[0:_KSEL, :], wu, preferred_element_type=f32)
                  + bur_ref[...])
    w8 = tv[0:_KSEL, :] / denom
    ctx = jnp.sum(w8 * u8, axis=0, keepdims=True)
    inp = jnp.concatenate([ctx, fb_ref[0]], axis=1)
    xh = jnp.concatenate([inp, m0r], axis=1)
    zg = jax.nn.sigmoid(jnp.dot(xh, wz_ref[...], preferred_element_type=f32)
                        + bz_ref[...])
    rg = jax.nn.sigmoid(jnp.dot(xh, wr_ref[...], preferred_element_type=f32)
                        + br_ref[...])
    xrh = jnp.concatenate([inp, rg * m0r], axis=1)
    hh = jnp.tanh(jnp.dot(xrh, wh_ref[...], preferred_element_type=f32)
                  + bh_ref[...])
    m1 = (1.0 - zg) * m0r + zg * hh
    mem_ref[0] = jnp.concatenate([m0r, m1], axis=0)


def kernel(x, feedback, params):
    p = params
    B, T, _ = x.shape
    f32 = jnp.float32

    full = lambda shape: pl.BlockSpec(shape, lambda b: tuple(0 for _ in shape))
    in_specs = [
        pl.BlockSpec((1, T, _DIN), lambda b: (b, 0, 0)),
        pl.BlockSpec((1, 1, 1), lambda b: (b, 0, 0)),
        full((_DIN, _DIN)), full((1, _DIN)), full((_DIN, 1)),
        full((_DIN, _DA)), full((_DA, 1)),
        full((_DA, _DM)), full((1, _DM)),
        full((_DA, 1)), full((1, 1)), full((_R, _Q)),
        full((_DM * 2 + 1, _DM)), full((1, _DM)),
        full((_DM * 2 + 1, _DM)), full((1, _DM)),
        full((_DM * 2 + 1, _DM)), full((1, _DM)),
        full((1, _DIN + 3)), full((1, _DIN + 3)),
        full((_DIN + 3, _KLAT)), full((1, _KLAT)),
        full((_KLAT, _DMODEL)), full((1, _DMODEL)),
    ]
    out_specs = (
        pl.BlockSpec((1, _R, _Q), lambda b: (b, 0, 0)),
        pl.BlockSpec((1, _KEFF, _DMODEL), lambda b: (b, 0, 0)),
        pl.BlockSpec((1, 2, _DM), lambda b: (b, 0, 0)),
    )
    all_y, tokens, mem = pl.pallas_call(
        _body,
        grid=(B,),
        in_specs=in_specs,
        out_specs=out_specs,
        out_shape=(
            jax.ShapeDtypeStruct((B, _R, _Q), f32),
            jax.ShapeDtypeStruct((B, _KEFF, _DMODEL), f32),
            jax.ShapeDtypeStruct((B, 2, _DM), f32),
        ),
    )(
        x, feedback.reshape(B, 1, 1),
        p['W_u'], p['b_u'].reshape(1, -1), p['b_u'].reshape(-1, 1),
        p['W_a'], p['b_a'].reshape(-1, 1),
        p['W_ma'].T, p['m0'].reshape(1, -1),
        p['w_s'].reshape(-1, 1), p['b_s'].reshape(1, 1),
        p['pos_bias'][:T].reshape(_R, _Q),
        p['W_z'], p['b_z'].reshape(1, -1),
        p['W_r'], p['b_r'].reshape(1, -1),
        p['W_h'], p['b_h'].reshape(1, -1),
        p['mu'].reshape(1, -1), p['sigma'].reshape(1, -1),
        p['W_lift'], p['b_lift'].reshape(1, -1),
        p['W_proj'], p['b_proj'].reshape(1, -1),
    )
    y_star = all_y.reshape(B, T)
    all_y = y_star[:, None, :]
    return tokens, y_star, all_y, mem


# speedup vs baseline: 1.0375x; 1.0104x over previous
import jax
import jax.numpy as jnp
from jax.experimental import pallas as pl

_B, _T = 4, 8192
_R = 8
_Q = _T // _R
_DM, _KLAT, _DMODEL = 64, 16, 128
_DIN, _DA = 64, 32
_KSEL, _KEFF = 8, 16
_PB = 2
_NEG = -1e30


def _body(x_ref, fb_ref, wu_ref, bur_ref, buc_ref, wa_ref, bac_ref, wmat_ref,
          m0r_ref, wsc_ref, bs_ref, pos_ref, wz_ref, bz_ref, wr_ref, br_ref,
          wh_ref, bh_ref, mu_ref, sig_ref, wl_ref, bl_ref, wp_ref, bp_ref,
          y_ref, tok_ref, mem_ref):
    f32 = jnp.float32
    i32 = jnp.int32
    wu = wu_ref[...]
    m0r = m0r_ref[...]
    wsc = wsc_ref[...]
    mwa_c = jnp.sum(wmat_ref[...] * m0r, axis=1, keepdims=True)
    iota_l = jax.lax.broadcasted_iota(i32, (1, _T), 1)

    dn_t = (((0,), (1,)), ((), ()))
    dn_tt = (((0,), (0,)), ((), ()))

    for bi in range(_PB):
        _one(bi, x_ref, fb_ref, wu, m0r, wsc, mwa_c, iota_l, dn_t, dn_tt,
             buc_ref, wa_ref, bac_ref, bs_ref, pos_ref, wz_ref, bz_ref,
             wr_ref, br_ref, wh_ref, bh_ref, mu_ref, sig_ref, wl_ref,
             bl_ref, wp_ref, bp_ref, bur_ref, y_ref, tok_ref, mem_ref)


def _one(bi, x_ref, fb_ref, wu, m0r, wsc, mwa_c, iota_l, dn_t, dn_tt,
         buc_ref, wa_ref, bac_ref, bs_ref, pos_ref, wz_ref, bz_ref,
         wr_ref, br_ref, wh_ref, bh_ref, mu_ref, sig_ref, wl_ref,
         bl_ref, wp_ref, bp_ref, bur_ref, y_ref, tok_ref, mem_ref):
    f32 = jnp.float32
    i32 = jnp.int32
    xb = x_ref[bi]
    ut = jnp.tanh(
        jax.lax.dot_general(wu, xb, dn_t, preferred_element_type=f32)
        + buc_ref[...])
    at = jnp.tanh(
        jax.lax.dot_general(wa_ref[...], ut, dn_tt, preferred_element_type=f32)
        + mwa_c + bac_ref[...])
    s8 = jnp.concatenate(
        [jnp.sum(at[:, _Q * i:_Q * (i + 1)] * wsc, axis=0, keepdims=True)
         for i in range(_R)], axis=0) + bs_ref[...] + pos_ref[...]
    ig = (_Q * jax.lax.broadcasted_iota(i32, (_R, _Q), 0)
          + jax.lax.broadcasted_iota(i32, (_R, _Q), 1))
    maxs = jnp.max(s8, keepdims=True)
    p_full = jnp.exp(s8 - maxs)
    sumexp = jnp.sum(p_full, keepdims=True)

    ms = s8
    selmask = jnp.zeros(s8.shape, jnp.bool_)
    pidxs, pjs = [], []
    for _ in range(_KSEL):
        v = jnp.max(ms, keepdims=True)
        pidx = jnp.min(jnp.where(ms == v, ig, _T), keepdims=True)
        pjs.append(jnp.exp(v - maxs) / sumexp)
        pidxs.append(pidx)
        selmask = selmask | (ig == pidx)
        ms = jnp.where(jnp.abs(ig - pidx) <= 1, _NEG, ms)

    p8r = jnp.concatenate(pjs, axis=1)
    i8r = jnp.concatenate(pidxs, axis=1)
    p8c = jnp.concatenate(pjs, axis=0)
    i8c = jnp.concatenate(pidxs, axis=0)

    y_ref[bi] = jnp.where(selmask, p_full / sumexp, 0.0)
    sump = jnp.sum(p8r, keepdims=True)
    denom = sump + 1e-8

    before = (p8c > p8r) | ((p8c == p8r) & (i8c < i8r))
    rank = jnp.sum(before.astype(i32), axis=0, keepdims=True)
    k8c = jax.lax.broadcasted_iota(i32, (_KSEL, 1), 0)
    mrank = (rank == k8c).astype(f32)
    svals = jnp.sum(mrank * p8r, axis=1, keepdims=True)
    sidx = jnp.sum(mrank.astype(i32) * i8r, axis=1, keepdims=True)
    t16r = jax.lax.broadcasted_iota(i32, (1, 2 * _KSEL), 1)
    picked = jnp.zeros((1, 2 * _KSEL), jnp.bool_)
    for pidx in pidxs:
        picked = picked | (t16r == pidx)
    free = ~picked
    t16c = jax.lax.broadcasted_iota(i32, (2 * _KSEL, 1), 0)
    free_c = jnp.sum((t16c == t16r).astype(i32)
                     * free.astype(i32), axis=1, keepdims=True)
    bc = jnp.sum(jnp.where((t16c < t16r) & (free_c > 0), 1, 0),
                 axis=0, keepdims=True)
    m2 = ((bc == k8c) & free).astype(i32)
    zidx = jnp.sum(m2 * t16r, axis=1, keepdims=True)
    tii = jnp.concatenate([sidx, zidx], axis=0)
    tv = jnp.concatenate([svals, jnp.zeros((_KSEL, 1), f32)], axis=0)

    sel17 = jnp.concatenate(
        [(tii == iota_l).astype(f32), jnp.ones((1, _T), f32)], axis=0)
    g17 = jnp.dot(sel17, xb, preferred_element_type=f32)
    xg = g17[0:_KEFF, :]
    xmean = g17[_KEFF:_KEFF + 1, :] * (1.0 / _T)

    i8f = i8r.astype(f32)
    cn = jnp.sum(p8r * (i8r <= tii).astype(f32), axis=1,
                 keepdims=True) / denom
    mean_cn = jnp.sum(p8r * (_T - i8f), keepdims=True) / (denom * _T)

    posn = tii.astype(f32) * (1.0 / _T)
    dvec = jnp.concatenate([xg, tv, posn, cn], axis=1)
    mp = jnp.full((1, 1), (_T - 1) / (2.0 * _T), f32)
    dmean = jnp.concatenate(
        [xmean, sump * (1.0 / _T), mp, mean_cn], axis=1)
    c = dvec - dmean
    c = c / (jnp.sqrt(jnp.sum(c * c, axis=1, keepdims=True)) + 1e-6)
    zz = (c - mu_ref[...]) / sig_ref[...]
    lif = jnp.tanh(jnp.dot(zz, wl_ref[...], preferred_element_type=f32)
                   + bl_ref[...])
    lif = lif / (jnp.sqrt(jnp.sum(lif * lif, axis=1, keepdims=True)) + 1e-6)
    tok_ref[bi] = (jnp.dot(lif, wp_ref[...], preferred_element_type=f32)
                  + bp_ref[...])

    u8 = jnp.tanh(jnp.dot(xg[0:_KSEL, :], wu, preferred_element_type=f32)
                  + bur_ref[...])
    w8 = tv[0:_KSEL, :] / denom
    ctx = jnp.sum(w8 * u8, axis=0, keepdims=True)
    inp = jnp.concatenate([ctx, fb_ref[bi]], axis=1)
    xh = jnp.concatenate([inp, m0r], axis=1)
    zg = jax.nn.sigmoid(jnp.dot(xh, wz_ref[...], preferred_element_type=f32)
                        + bz_ref[...])
    rg = jax.nn.sigmoid(jnp.dot(xh, wr_ref[...], preferred_element_type=f32)
                        + br_ref[...])
    xrh = jnp.concatenate([inp, rg * m0r], axis=1)
    hh = jnp.tanh(jnp.dot(xrh, wh_ref[...], preferred_element_type=f32)
                  + bh_ref[...])
    m1 = (1.0 - zg) * m0r + zg * hh
    mem_ref[bi] = jnp.concatenate([m0r, m1], axis=0)


def kernel(x, feedback, params):
    p = params
    B, T, _ = x.shape
    f32 = jnp.float32

    full = lambda shape: pl.BlockSpec(shape, lambda b: tuple(0 for _ in shape))
    in_specs = [
        pl.BlockSpec((_PB, T, _DIN), lambda b: (b, 0, 0)),
        pl.BlockSpec((_PB, 1, 1), lambda b: (b, 0, 0)),
        full((_DIN, _DIN)), full((1, _DIN)), full((_DIN, 1)),
        full((_DIN, _DA)), full((_DA, 1)),
        full((_DA, _DM)), full((1, _DM)),
        full((_DA, 1)), full((1, 1)), full((_R, _Q)),
        full((_DM * 2 + 1, _DM)), full((1, _DM)),
        full((_DM * 2 + 1, _DM)), full((1, _DM)),
        full((_DM * 2 + 1, _DM)), full((1, _DM)),
        full((1, _DIN + 3)), full((1, _DIN + 3)),
        full((_DIN + 3, _KLAT)), full((1, _KLAT)),
        full((_KLAT, _DMODEL)), full((1, _DMODEL)),
    ]
    out_specs = (
        pl.BlockSpec((_PB, _R, _Q), lambda b: (b, 0, 0)),
        pl.BlockSpec((_PB, _KEFF, _DMODEL), lambda b: (b, 0, 0)),
        pl.BlockSpec((_PB, 2, _DM), lambda b: (b, 0, 0)),
    )
    all_y, tokens, mem = pl.pallas_call(
        _body,
        grid=(B // _PB,),
        in_specs=in_specs,
        out_specs=out_specs,
        out_shape=(
            jax.ShapeDtypeStruct((B, _R, _Q), f32),
            jax.ShapeDtypeStruct((B, _KEFF, _DMODEL), f32),
            jax.ShapeDtypeStruct((B, 2, _DM), f32),
        ),
    )(
        x, feedback.reshape(B, 1, 1),
        p['W_u'], p['b_u'].reshape(1, -1), p['b_u'].reshape(-1, 1),
        p['W_a'], p['b_a'].reshape(-1, 1),
        p['W_ma'].T, p['m0'].reshape(1, -1),
        p['w_s'].reshape(-1, 1), p['b_s'].reshape(1, 1),
        p['pos_bias'][:T].reshape(_R, _Q),
        p['W_z'], p['b_z'].reshape(1, -1),
        p['W_r'], p['b_r'].reshape(1, -1),
        p['W_h'], p['b_h'].reshape(1, -1),
        p['mu'].reshape(1, -1), p['sigma'].reshape(1, -1),
        p['W_lift'], p['b_lift'].reshape(1, -1),
        p['W_proj'], p['b_proj'].reshape(1, -1),
    )
    y_star = all_y.reshape(B, T)
    all_y = y_star[:, None, :]
    return tokens, y_star, all_y, mem


# speedup vs baseline: 1.0835x; 1.0443x over previous
import jax
import jax.numpy as jnp
from jax.experimental import pallas as pl

_B, _T = 4, 8192
_R = 8
_Q = _T // _R
_DM, _KLAT, _DMODEL = 64, 16, 128
_DIN, _DA = 64, 32
_KSEL, _KEFF = 8, 16
_PB = 2
_NEG = -1e30

_PK_BU, _PK_BA, _PK_WS, _PK_BS = 0, 128, 256, 384
_PK_M0, _PK_BZ, _PK_BR, _PK_BH = 512, 640, 768, 896
_PK_MU, _PK_SG, _PK_BL, _PK_BP = 1024, 1152, 1280, 1408
_PK_LEN = 1536


def _col(row, n):
    i32 = jnp.int32
    eq = (jax.lax.broadcasted_iota(i32, (n, n), 0)
          == jax.lax.broadcasted_iota(i32, (n, n), 1))
    return jnp.sum(jnp.where(eq, row, 0.0), axis=1, keepdims=True)


def _body(x_ref, fb_ref, wu_ref, wa_ref, wma_ref, wz_ref, wr_ref, wh_ref,
          wl_ref, wp_ref, vp_ref, pos_ref, y_ref, tok_ref, mem_ref):
    f32 = jnp.float32
    i32 = jnp.int32
    vp = vp_ref[...]
    bu_row = vp[:, _PK_BU:_PK_BU + _DIN]
    ba_row = vp[:, _PK_BA:_PK_BA + _DA]
    ws_row = vp[:, _PK_WS:_PK_WS + _DA]
    bs = vp[:, _PK_BS:_PK_BS + 1]
    m0r = vp[:, _PK_M0:_PK_M0 + _DM]
    bz = vp[:, _PK_BZ:_PK_BZ + _DM]
    br = vp[:, _PK_BR:_PK_BR + _DM]
    bh = vp[:, _PK_BH:_PK_BH + _DM]
    mu = vp[:, _PK_MU:_PK_MU + _DIN + 3]
    sig = vp[:, _PK_SG:_PK_SG + _DIN + 3]
    blift = vp[:, _PK_BL:_PK_BL + _KLAT]
    bproj = vp[:, _PK_BP:_PK_BP + _DMODEL]

    wu = wu_ref[...]
    buc = _col(bu_row, _DIN)
    bac = _col(ba_row, _DA)
    wsc = _col(ws_row, _DA)
    mwa_row = jnp.dot(m0r, wma_ref[...], preferred_element_type=f32)
    mwa_c = _col(mwa_row, _DA)
    iota_l = jax.lax.broadcasted_iota(i32, (1, _T), 1)

    dn_t = (((0,), (1,)), ((), ()))
    dn_tt = (((0,), (0,)), ((), ()))

    for bi in range(_PB):
        xb = x_ref[bi]
        ut = jnp.tanh(
            jax.lax.dot_general(wu, xb, dn_t, preferred_element_type=f32)
            + buc)
        at = jnp.tanh(
            jax.lax.dot_general(wa_ref[...], ut, dn_tt,
                                preferred_element_type=f32)
            + mwa_c + bac)
        s8 = jnp.concatenate(
            [jnp.sum(at[:, _Q * i:_Q * (i + 1)] * wsc, axis=0, keepdims=True)
             for i in range(_R)], axis=0) + bs + pos_ref[...]
        ig = (_Q * jax.lax.broadcasted_iota(i32, (_R, _Q), 0)
              + jax.lax.broadcasted_iota(i32, (_R, _Q), 1))
        maxs = jnp.max(s8, keepdims=True)
        sumexp = jnp.sum(jnp.exp(s8 - maxs), keepdims=True)

        ms = s8
        pidxs, pjs = [], []
        for _ in range(_KSEL):
            v = jnp.max(ms, keepdims=True)
            pidx = jnp.min(jnp.where(ms == v, ig, _T), keepdims=True)
            pjs.append(jnp.exp(v - maxs) / sumexp)
            pidxs.append(pidx)
            ms = jnp.where(jnp.abs(ig - pidx) <= 1, _NEG, ms)

        p8r = jnp.concatenate(pjs, axis=1)
        i8r = jnp.concatenate(pidxs, axis=1)
        p8c = jnp.concatenate(pjs, axis=0)
        i8c = jnp.concatenate(pidxs, axis=0)

        y_row = jnp.zeros((1, _T), f32)
        for pidx, pj in zip(pidxs, pjs):
            y_row = jnp.where(iota_l == pidx, pj, y_row)
        y_ref[bi] = y_row
        sump = jnp.sum(p8r, keepdims=True)
        denom = sump + 1e-8

        before = (p8c > p8r) | ((p8c == p8r) & (i8c < i8r))
        rank = jnp.sum(before.astype(i32), axis=0, keepdims=True)
        k8c = jax.lax.broadcasted_iota(i32, (_KSEL, 1), 0)
        mrank = (rank == k8c).astype(f32)
        svals = jnp.sum(mrank * p8r, axis=1, keepdims=True)
        sidx = jnp.sum(mrank.astype(i32) * i8r, axis=1, keepdims=True)
        t16r = jax.lax.broadcasted_iota(i32, (1, 2 * _KSEL), 1)
        picked = jnp.zeros((1, 2 * _KSEL), jnp.bool_)
        for pidx in pidxs:
            picked = picked | (t16r == pidx)
        free = ~picked
        t16c = jax.lax.broadcasted_iota(i32, (2 * _KSEL, 1), 0)
        free_c = jnp.sum((t16c == t16r).astype(i32)
                         * free.astype(i32), axis=1, keepdims=True)
        bc = jnp.sum(jnp.where((t16c < t16r) & (free_c > 0), 1, 0),
                     axis=0, keepdims=True)
        m2 = ((bc == k8c) & free).astype(i32)
        zidx = jnp.sum(m2 * t16r, axis=1, keepdims=True)
        tii = jnp.concatenate([sidx, zidx], axis=0)
        tv = jnp.concatenate([svals, jnp.zeros((_KSEL, 1), f32)], axis=0)

        sel17 = jnp.concatenate(
            [(tii == iota_l).astype(f32), jnp.ones((1, _T), f32)], axis=0)
        g17 = jnp.dot(sel17, xb, preferred_element_type=f32)
        xg = g17[0:_KEFF, :]
        xmean = g17[_KEFF:_KEFF + 1, :] * (1.0 / _T)

        i8f = i8r.astype(f32)
        cn = jnp.sum(p8r * (i8r <= tii).astype(f32), axis=1,
                     keepdims=True) / denom
        mean_cn = jnp.sum(p8r * (_T - i8f), keepdims=True) / (denom * _T)

        posn = tii.astype(f32) * (1.0 / _T)
        dvec = jnp.concatenate([xg, tv, posn, cn], axis=1)
        mp = jnp.full((1, 1), (_T - 1) / (2.0 * _T), f32)
        dmean = jnp.concatenate(
            [xmean, sump * (1.0 / _T), mp, mean_cn], axis=1)
        c = dvec - dmean
        c = c / (jnp.sqrt(jnp.sum(c * c, axis=1, keepdims=True)) + 1e-6)
        zz = (c - mu) / sig
        lif = jnp.tanh(jnp.dot(zz, wl_ref[...], preferred_element_type=f32)
                       + blift)
        lif = lif / (jnp.sqrt(jnp.sum(lif * lif, axis=1, keepdims=True))
                     + 1e-6)
        tok_ref[bi] = (jnp.dot(lif, wp_ref[...], preferred_element_type=f32)
                       + bproj)

        u8 = jnp.tanh(jnp.dot(xg[0:_KSEL, :], wu, preferred_element_type=f32)
                      + bu_row)
        w8 = tv[0:_KSEL, :] / denom
        ctx = jnp.sum(w8 * u8, axis=0, keepdims=True)
        brow = _PB * pl.program_id(0) + bi
        iota_b = jax.lax.broadcasted_iota(i32, (_B, 1), 0)
        fbb = jnp.sum(jnp.where(iota_b == brow, fb_ref[...], 0.0),
                      axis=0, keepdims=True)
        inp = jnp.concatenate([ctx, fbb], axis=1)
        xh = jnp.concatenate([inp, m0r], axis=1)
        zg = jax.nn.sigmoid(jnp.dot(xh, wz_ref[...],
                                    preferred_element_type=f32) + bz)
        rg = jax.nn.sigmoid(jnp.dot(xh, wr_ref[...],
                                    preferred_element_type=f32) + br)
        xrh = jnp.concatenate([inp, rg * m0r], axis=1)
        hh = jnp.tanh(jnp.dot(xrh, wh_ref[...], preferred_element_type=f32)
                      + bh)
        m1 = (1.0 - zg) * m0r + zg * hh
        mem_ref[bi] = jnp.concatenate([m0r, m1], axis=0)


def kernel(x, feedback, params):
    p = params
    B, T, _ = x.shape
    f32 = jnp.float32
    z = jnp.zeros
    vpack = jnp.concatenate([
        p['b_u'], z(64, f32),
        p['b_a'], z(96, f32),
        p['w_s'], z(96, f32),
        p['b_s'].reshape(1), z(127, f32),
        p['m0'], z(64, f32),
        p['b_z'], z(64, f32),
        p['b_r'], z(64, f32),
        p['b_h'], z(64, f32),
        p['mu'], z(61, f32),
        p['sigma'], z(61, f32),
        p['b_lift'], z(112, f32),
        p['b_proj'],
    ]).reshape(1, _PK_LEN)
    pos8 = p['pos_bias'][:T].reshape(_R, _Q)

    full = lambda shape: pl.BlockSpec(shape, lambda b: tuple(0 for _ in shape))
    in_specs = [
        pl.BlockSpec((_PB, T, _DIN), lambda b: (b, 0, 0)),
        full((B, 1)),
        full((_DIN, _DIN)), full((_DIN, _DA)), full((_DM, _DA)),
        full((_DM * 2 + 1, _DM)), full((_DM * 2 + 1, _DM)),
        full((_DM * 2 + 1, _DM)),
        full((_DIN + 3, _KLAT)), full((_KLAT, _DMODEL)),
        full((1, _PK_LEN)), full((_R, _Q)),
    ]
    out_specs = (
        pl.BlockSpec((_PB, 1, T), lambda b: (b, 0, 0)),
        pl.BlockSpec((_PB, _KEFF, _DMODEL), lambda b: (b, 0, 0)),
        pl.BlockSpec((_PB, 2, _DM), lambda b: (b, 0, 0)),
    )
    all_y, tokens, mem = pl.pallas_call(
        _body,
        grid=(B // _PB,),
        in_specs=in_specs,
        out_specs=out_specs,
        out_shape=(
            jax.ShapeDtypeStruct((B, 1, T), f32),
            jax.ShapeDtypeStruct((B, _KEFF, _DMODEL), f32),
            jax.ShapeDtypeStruct((B, 2, _DM), f32),
        ),
    )(
        x, feedback,
        p['W_u'], p['W_a'], p['W_ma'],
        p['W_z'], p['W_r'], p['W_h'],
        p['W_lift'], p['W_proj'],
        vpack, pos8,
    )
    y_star = all_y[:, 0, :]
    return tokens, y_star, all_y, mem


# speedup vs baseline: 1.1040x; 1.0189x over previous
import jax
import jax.numpy as jnp
from jax.experimental import pallas as pl

_B, _T = 4, 8192
_R = 8
_Q = _T // _R
_DM, _KLAT, _DMODEL = 64, 16, 128
_DIN, _DA = 64, 32
_KSEL, _KEFF = 8, 16
_PB = 1
_NEG = -1e30

_PK_BU, _PK_BA, _PK_WS, _PK_BS = 0, 128, 256, 384
_PK_M0, _PK_BZ, _PK_BR, _PK_BH = 512, 640, 768, 896
_PK_MU, _PK_SG, _PK_BL, _PK_BP = 1024, 1152, 1280, 1408
_PK_LEN = 1536


def _col(row, n):
    i32 = jnp.int32
    eq = (jax.lax.broadcasted_iota(i32, (n, n), 0)
          == jax.lax.broadcasted_iota(i32, (n, n), 1))
    return jnp.sum(jnp.where(eq, row, 0.0), axis=1, keepdims=True)


def _body(x_ref, fb_ref, wu_ref, wa_ref, wma_ref, wz_ref, wr_ref, wh_ref,
          wl_ref, wp_ref, vp_ref, pos_ref, y_ref, tok_ref, mem_ref):
    f32 = jnp.float32
    i32 = jnp.int32
    vp = vp_ref[...]
    bu_row = vp[:, _PK_BU:_PK_BU + _DIN]
    ba_row = vp[:, _PK_BA:_PK_BA + _DA]
    ws_row = vp[:, _PK_WS:_PK_WS + _DA]
    bs = vp[:, _PK_BS:_PK_BS + 1]
    m0r = vp[:, _PK_M0:_PK_M0 + _DM]
    bz = vp[:, _PK_BZ:_PK_BZ + _DM]
    br = vp[:, _PK_BR:_PK_BR + _DM]
    bh = vp[:, _PK_BH:_PK_BH + _DM]
    mu = vp[:, _PK_MU:_PK_MU + _DIN + 3]
    sig = vp[:, _PK_SG:_PK_SG + _DIN + 3]
    blift = vp[:, _PK_BL:_PK_BL + _KLAT]
    bproj = vp[:, _PK_BP:_PK_BP + _DMODEL]

    wu = wu_ref[...]
    buc = _col(bu_row, _DIN)
    bac = _col(ba_row, _DA)
    wsc = _col(ws_row, _DA)
    mwa_row = jnp.dot(m0r, wma_ref[...], preferred_element_type=f32)
    mwa_c = _col(mwa_row, _DA)
    iota_l = jax.lax.broadcasted_iota(i32, (1, _T), 1)

    dn_t = (((0,), (1,)), ((), ()))
    dn_tt = (((0,), (0,)), ((), ()))

    for bi in range(_PB):
        xb = x_ref[bi]
        ut = jnp.tanh(
            jax.lax.dot_general(wu, xb, dn_t, preferred_element_type=f32)
            + buc)
        at = jnp.tanh(
            jax.lax.dot_general(wa_ref[...], ut, dn_tt,
                                preferred_element_type=f32)
            + mwa_c + bac)
        s8 = jnp.concatenate(
            [jnp.sum(at[:, _Q * i:_Q * (i + 1)] * wsc, axis=0, keepdims=True)
             for i in range(_R)], axis=0) + bs + pos_ref[...]
        ig = (_Q * jax.lax.broadcasted_iota(i32, (_R, _Q), 0)
              + jax.lax.broadcasted_iota(i32, (_R, _Q), 1))
        maxs = jnp.max(s8, keepdims=True)
        sumexp = jnp.sum(jnp.exp(s8 - maxs), keepdims=True)

        ms = s8
        pidxs, pjs = [], []
        for _ in range(_KSEL):
            v = jnp.max(ms, keepdims=True)
            pidx = jnp.min(jnp.where(ms == v, ig, _T), keepdims=True)
            pjs.append(jnp.exp(v - maxs) / sumexp)
            pidxs.append(pidx)
            ms = jnp.where(jnp.abs(ig - pidx) <= 1, _NEG, ms)

        p8r = jnp.concatenate(pjs, axis=1)
        i8r = jnp.concatenate(pidxs, axis=1)
        p8c = jnp.concatenate(pjs, axis=0)
        i8c = jnp.concatenate(pidxs, axis=0)

        y_row = jnp.zeros((1, _T), f32)
        for pidx, pj in zip(pidxs, pjs):
            y_row = jnp.where(iota_l == pidx, pj, y_row)
        y_ref[bi] = y_row
        sump = jnp.sum(p8r, keepdims=True)
        denom = sump + 1e-8

        before = (p8c > p8r) | ((p8c == p8r) & (i8c < i8r))
        rank = jnp.sum(before.astype(i32), axis=0, keepdims=True)
        k8c = jax.lax.broadcasted_iota(i32, (_KSEL, 1), 0)
        mrank = (rank == k8c).astype(f32)
        svals = jnp.sum(mrank * p8r, axis=1, keepdims=True)
        sidx = jnp.sum(mrank.astype(i32) * i8r, axis=1, keepdims=True)
        t16r = jax.lax.broadcasted_iota(i32, (1, 2 * _KSEL), 1)
        picked = jnp.zeros((1, 2 * _KSEL), jnp.bool_)
        for pidx in pidxs:
            picked = picked | (t16r == pidx)
        free = ~picked
        t16c = jax.lax.broadcasted_iota(i32, (2 * _KSEL, 1), 0)
        free_c = jnp.sum((t16c == t16r).astype(i32)
                         * free.astype(i32), axis=1, keepdims=True)
        bc = jnp.sum(jnp.where((t16c < t16r) & (free_c > 0), 1, 0),
                     axis=0, keepdims=True)
        m2 = ((bc == k8c) & free).astype(i32)
        zidx = jnp.sum(m2 * t16r, axis=1, keepdims=True)
        tii = jnp.concatenate([sidx, zidx], axis=0)
        tv = jnp.concatenate([svals, jnp.zeros((_KSEL, 1), f32)], axis=0)

        sel17 = jnp.concatenate(
            [(tii == iota_l).astype(f32), jnp.ones((1, _T), f32)], axis=0)
        g17 = jnp.dot(sel17, xb, preferred_element_type=f32)
        xg = g17[0:_KEFF, :]
        xmean = g17[_KEFF:_KEFF + 1, :] * (1.0 / _T)

        i8f = i8r.astype(f32)
        cn = jnp.sum(p8r * (i8r <= tii).astype(f32), axis=1,
                     keepdims=True) / denom
        mean_cn = jnp.sum(p8r * (_T - i8f), keepdims=True) / (denom * _T)

        posn = tii.astype(f32) * (1.0 / _T)
        dvec = jnp.concatenate([xg, tv, posn, cn], axis=1)
        mp = jnp.full((1, 1), (_T - 1) / (2.0 * _T), f32)
        dmean = jnp.concatenate(
            [xmean, sump * (1.0 / _T), mp, mean_cn], axis=1)
        c = dvec - dmean
        c = c / (jnp.sqrt(jnp.sum(c * c, axis=1, keepdims=True)) + 1e-6)
        zz = (c - mu) / sig
        lif = jnp.tanh(jnp.dot(zz, wl_ref[...], preferred_element_type=f32)
                       + blift)
        lif = lif / (jnp.sqrt(jnp.sum(lif * lif, axis=1, keepdims=True))
                     + 1e-6)
        tok_ref[bi] = (jnp.dot(lif, wp_ref[...], preferred_element_type=f32)
                       + bproj)

        u8 = jnp.tanh(jnp.dot(xg[0:_KSEL, :], wu, preferred_element_type=f32)
                      + bu_row)
        w8 = tv[0:_KSEL, :] / denom
        ctx = jnp.sum(w8 * u8, axis=0, keepdims=True)
        brow = _PB * pl.program_id(0) + bi
        iota_b = jax.lax.broadcasted_iota(i32, (_B, 1), 0)
        fbb = jnp.sum(jnp.where(iota_b == brow, fb_ref[...], 0.0),
                      axis=0, keepdims=True)
        inp = jnp.concatenate([ctx, fbb], axis=1)
        xh = jnp.concatenate([inp, m0r], axis=1)
        zg = jax.nn.sigmoid(jnp.dot(xh, wz_ref[...],
                                    preferred_element_type=f32) + bz)
        rg = jax.nn.sigmoid(jnp.dot(xh, wr_ref[...],
                                    preferred_element_type=f32) + br)
        xrh = jnp.concatenate([inp, rg * m0r], axis=1)
        hh = jnp.tanh(jnp.dot(xrh, wh_ref[...], preferred_element_type=f32)
                      + bh)
        m1 = (1.0 - zg) * m0r + zg * hh
        mem_ref[bi] = jnp.concatenate([m0r, m1], axis=0)


def kernel(x, feedback, params):
    p = params
    B, T, _ = x.shape
    f32 = jnp.float32
    z = jnp.zeros
    vpack = jnp.concatenate([
        p['b_u'], z(64, f32),
        p['b_a'], z(96, f32),
        p['w_s'], z(96, f32),
        p['b_s'].reshape(1), z(127, f32),
        p['m0'], z(64, f32),
        p['b_z'], z(64, f32),
        p['b_r'], z(64, f32),
        p['b_h'], z(64, f32),
        p['mu'], z(61, f32),
        p['sigma'], z(61, f32),
        p['b_lift'], z(112, f32),
        p['b_proj'],
    ]).reshape(1, _PK_LEN)
    pos8 = p['pos_bias'][:T].reshape(_R, _Q)

    full = lambda shape: pl.BlockSpec(shape, lambda b: tuple(0 for _ in shape))
    in_specs = [
        pl.BlockSpec((_PB, T, _DIN), lambda b: (b, 0, 0)),
        full((B, 1)),
        full((_DIN, _DIN)), full((_DIN, _DA)), full((_DM, _DA)),
        full((_DM * 2 + 1, _DM)), full((_DM * 2 + 1, _DM)),
        full((_DM * 2 + 1, _DM)),
        full((_DIN + 3, _KLAT)), full((_KLAT, _DMODEL)),
        full((1, _PK_LEN)), full((_R, _Q)),
    ]
    out_specs = (
        pl.BlockSpec((_PB, 1, T), lambda b: (b, 0, 0)),
        pl.BlockSpec((_PB, _KEFF, _DMODEL), lambda b: (b, 0, 0)),
        pl.BlockSpec((_PB, 2, _DM), lambda b: (b, 0, 0)),
    )
    all_y, tokens, mem = pl.pallas_call(
        _body,
        grid=(B // _PB,),
        in_specs=in_specs,
        out_specs=out_specs,
        out_shape=(
            jax.ShapeDtypeStruct((B, 1, T), f32),
            jax.ShapeDtypeStruct((B, _KEFF, _DMODEL), f32),
            jax.ShapeDtypeStruct((B, 2, _DM), f32),
        ),
    )(
        x, feedback,
        p['W_u'], p['W_a'], p['W_ma'],
        p['W_z'], p['W_r'], p['W_h'],
        p['W_lift'], p['W_proj'],
        vpack, pos8,
    )
    y_star = all_y[:, 0, :]
    return tokens, y_star, all_y, mem
